# Initial kernel scaffold; baseline (speedup 1.0000x reference)
import jax
import jax.numpy as jnp
from jax.experimental import pallas as pl


def kernel(output, target):
    raise NotImplementedError("write your pallas kernel here")


# speedup vs baseline: 53.5002x; 53.5002x over previous
import numpy as np
import jax
import jax.numpy as jnp
from jax.experimental import pallas as pl
from jax.experimental.pallas import tpu as pltpu

_N = 512
_B = 32
_RNYQ = _N // 2
_NB_PAD = 512
_EPS = 1e-8
_PCHUNK = 4096
_NP = (_N * _N) // _PCHUNK


def _build_consts():
    k = np.arange(_N, dtype=np.int64)
    ang = 2.0 * np.pi * ((np.outer(k, k) % _N).astype(np.float64)) / _N
    c = (np.cos(ang) / _N).astype(np.float32)
    s = (np.sin(ang) / _N).astype(np.float32)
    f = np.fft.fftfreq(_N, 1.0 / _N).astype(np.float32)
    fx, fy = np.meshgrid(f, f, indexing="ij")
    rad = np.round(np.sqrt(fx * fx + fy * fy)).astype(np.int32)
    idx = np.where(rad > _RNYQ, _RNYQ + 1, rad).astype(np.int32)
    return c, s, idx.reshape(_NP, 1, _PCHUNK)


_C_HOST, _S_HOST, _IDX_HOST = _build_consts()


def _dft_products_kernel(x1_ref, x2_ref, c_ref, s_ref, o_ref):
    x1 = x1_ref[0]
    x2 = x2_ref[0]
    c = c_ref[...]
    s = s_ref[...]

    def dot(a, b):
        return jax.lax.dot_general(
            a, b, (((1,), (0,)), ((), ())),
            preferred_element_type=jnp.float32)

    t1c = dot(x1, c)
    t1s = dot(x1, s)
    t2c = dot(x2, c)
    t2s = dot(x2, s)
    f1r = dot(c, t1c) - dot(s, t1s)
    f1i = -(dot(c, t1s) + dot(s, t1c))
    f2r = dot(c, t2c) - dot(s, t2s)
    f2i = -(dot(c, t2s) + dot(s, t2c))

    o_ref[0, 0] = f1r * f2r + f1i * f2i
    o_ref[1, 0] = f1i * f2r - f1r * f2i
    o_ref[2, 0] = f1r * f1r + f1i * f1i
    o_ref[3, 0] = f2r * f2r + f2i * f2i


def _bin_kernel(idx_ref, v_ref, o_ref):
    p = pl.program_id(1)

    @pl.when(p == 0)
    def _():
        o_ref[...] = jnp.zeros_like(o_ref)

    idxv = idx_ref[0]
    bins = jax.lax.broadcasted_iota(jnp.int32, (_NB_PAD, _PCHUNK), 0)
    onehot = jnp.where(idxv == bins, 1.0, 0.0)
    vals = v_ref[...].reshape(4 * _B, _PCHUNK)
    o_ref[0] += jax.lax.dot_general(
        onehot, vals, (((1,), (1,)), ((), ())),
        preferred_element_type=jnp.float32)


def _loss_kernel(s_ref, o_ref):
    s = s_ref[0] + s_ref[1]
    cr = s[:, 0 * _B:1 * _B]
    ci = s[:, 1 * _B:2 * _B]
    c1 = s[:, 2 * _B:3 * _B]
    c2 = s[:, 3 * _B:4 * _B]
    frc = jnp.sqrt(cr * cr + ci * ci) / (jnp.sqrt(c1 * c2) + _EPS)
    mask = jax.lax.broadcasted_iota(jnp.int32, (_NB_PAD, _B), 0) <= _RNYQ
    term = jnp.where(mask, (1.0 - frc) ** 2, 0.0)
    total = jnp.sum(term, axis=(0, 1), keepdims=True)
    o_ref[...] = total * (1.0 / ((_RNYQ + 1) * _B))


def kernel(output, target):
    x1 = output[:, 0]
    x2 = target[:, 0]
    c = jnp.asarray(_C_HOST)
    s = jnp.asarray(_S_HOST)
    idx = jnp.asarray(_IDX_HOST)

    prods = pl.pallas_call(
        _dft_products_kernel,
        grid=(_B,),
        in_specs=[
            pl.BlockSpec((1, _N, _N), lambda b: (b, 0, 0)),
            pl.BlockSpec((1, _N, _N), lambda b: (b, 0, 0)),
            pl.BlockSpec((_N, _N), lambda b: (0, 0)),
            pl.BlockSpec((_N, _N), lambda b: (0, 0)),
        ],
        out_specs=pl.BlockSpec((4, 1, _N, _N), lambda b: (0, b, 0, 0)),
        out_shape=jax.ShapeDtypeStruct((4, _B, _N, _N), jnp.float32),
        compiler_params=pltpu.CompilerParams(
            dimension_semantics=("parallel",),
            vmem_limit_bytes=50 * 1024 * 1024,
        ),
        name="frc_dft_products",
    )(x1, x2, c, s)

    prods_flat = prods.reshape(4, _B, _N * _N)
    half = _NP // 2
    part = pl.pallas_call(
        _bin_kernel,
        grid=(2, half),
        in_specs=[
            pl.BlockSpec((1, 1, _PCHUNK), lambda h, p: (h * half + p, 0, 0)),
            pl.BlockSpec((4, _B, _PCHUNK), lambda h, p: (0, 0, h * half + p)),
        ],
        out_specs=pl.BlockSpec((1, _NB_PAD, 4 * _B), lambda h, p: (h, 0, 0)),
        out_shape=jax.ShapeDtypeStruct((2, _NB_PAD, 4 * _B), jnp.float32),
        compiler_params=pltpu.CompilerParams(
            dimension_semantics=("parallel", "arbitrary"),
            vmem_limit_bytes=50 * 1024 * 1024,
        ),
        name="frc_radial_bins",
    )(idx, prods_flat)

    loss = pl.pallas_call(
        _loss_kernel,
        out_shape=jax.ShapeDtypeStruct((1, 1), jnp.float32),
        name="frc_loss_epilogue",
    )(part)
    return loss[0, 0]


# speedup vs baseline: 85.1868x; 1.5923x over previous
import numpy as np
import ml_dtypes
import jax
import jax.numpy as jnp
from jax.experimental import pallas as pl
from jax.experimental.pallas import tpu as pltpu

_N = 512
_B = 32
_RNYQ = _N // 2
_NB_PAD = 512
_EPS = 1e-8
_NCOL = 320
_NVALID = _RNYQ + 1
_PCHUNK = 4096
_NP = (_N * _NCOL) // _PCHUNK
_BF16 = ml_dtypes.bfloat16


def _build_consts():
    k = np.arange(_N, dtype=np.int64)
    ang = 2.0 * np.pi * ((np.outer(k, k) % _N).astype(np.float64)) / _N
    c64 = np.cos(ang) / _N
    s64 = np.sin(ang) / _N
    cf = c64.astype(_BF16)
    sf = s64.astype(_BF16)
    cms = (c64 - s64).astype(_BF16)
    ch = np.zeros((_N, _NCOL), dtype=_BF16)
    sh = np.zeros((_N, _NCOL), dtype=_BF16)
    ch[:, :_NVALID] = c64[:, :_NVALID].astype(_BF16)
    sh[:, :_NVALID] = s64[:, :_NVALID].astype(_BF16)

    f = np.fft.fftfreq(_N, 1.0 / _N).astype(np.float32)
    fx = f.reshape(_N, 1)
    fy = f[:_NVALID].reshape(1, _NVALID)
    rad = np.round(np.sqrt(fx * fx + fy * fy)).astype(np.int32)
    idx = np.full((_N, _NCOL), _RNYQ + 1, dtype=np.int32)
    idx[:, :_NVALID] = np.where(rad > _RNYQ, _RNYQ + 1, rad)
    w = np.zeros((_N, _NCOL), dtype=np.float32)
    w[:, 1:_RNYQ] = 2.0
    w[:, 0] = 1.0
    w[:, _RNYQ] = 1.0
    return (cf, sf, cms, ch, sh,
            idx.reshape(_NP, 1, _PCHUNK), w.reshape(_NP, 1, _PCHUNK))


(_CF_H, _SF_H, _CMS_H, _CH_H, _SH_H, _IDX_H, _W_H) = _build_consts()


def _dft_products_kernel(x1_ref, x2_ref, ch_ref, sh_ref, c_ref, s_ref,
                         cms_ref, o_ref):
    ch = ch_ref[...]
    sh = sh_ref[...]
    c = c_ref[...]
    s = s_ref[...]
    cms = cms_ref[...]

    def bdot(a, b):
        return jax.lax.dot_general(
            a, b, (((1,), (0,)), ((), ())),
            preferred_element_type=jnp.float32)

    def half_fft(x_ref):
        x = x_ref[0].astype(jnp.bfloat16)
        tr = bdot(x, ch)
        ns = bdot(x, sh)
        u = (tr - ns).astype(jnp.bfloat16)
        tr16 = tr.astype(jnp.bfloat16)
        ti16 = (-ns).astype(jnp.bfloat16)
        p = bdot(c, tr16)
        q = bdot(s, ti16)
        r = bdot(cms, u)
        return p + q, r - p + q

    f1r, f1i = half_fft(x1_ref)
    f2r, f2i = half_fft(x2_ref)

    o_ref[0, 0] = (f1r * f2r + f1i * f2i).astype(jnp.bfloat16)
    o_ref[1, 0] = (f1r * f1r + f1i * f1i).astype(jnp.bfloat16)
    o_ref[2, 0] = (f2r * f2r + f2i * f2i).astype(jnp.bfloat16)


def _bin_kernel(idx_ref, w_ref, v_ref, o_ref):
    p = pl.program_id(1)

    @pl.when(p == 0)
    def _():
        o_ref[...] = jnp.zeros_like(o_ref)

    idxv = idx_ref[0]
    wv = w_ref[0]
    bins = jax.lax.broadcasted_iota(jnp.int32, (_NB_PAD, _PCHUNK), 0)
    onehot = jnp.where(idxv == bins, wv, 0.0).astype(jnp.bfloat16)
    vals = v_ref[...].reshape(3 * _B, _PCHUNK)
    o_ref[0] += jax.lax.dot_general(
        vals, onehot, (((1,), (1,)), ((), ())),
        preferred_element_type=jnp.float32)


def _loss_kernel(s_ref, o_ref):
    s = s_ref[0] + s_ref[1]
    cr = s[0 * _B:1 * _B, :]
    c1 = s[1 * _B:2 * _B, :]
    c2 = s[2 * _B:3 * _B, :]
    frc = jnp.abs(cr) / (jnp.sqrt(c1 * c2) + _EPS)
    mask = jax.lax.broadcasted_iota(jnp.int32, (_B, _NB_PAD), 1) <= _RNYQ
    term = jnp.where(mask, (1.0 - frc) ** 2, 0.0)
    total = jnp.sum(term, axis=(0, 1), keepdims=True)
    o_ref[...] = total * (1.0 / (_NVALID * _B))


def kernel(output, target):
    x1 = output[:, 0]
    x2 = target[:, 0]
    cf = jnp.asarray(_CF_H)
    sf = jnp.asarray(_SF_H)
    cms = jnp.asarray(_CMS_H)
    chm = jnp.asarray(_CH_H)
    shm = jnp.asarray(_SH_H)
    idx = jnp.asarray(_IDX_H)
    w = jnp.asarray(_W_H)

    prods = pl.pallas_call(
        _dft_products_kernel,
        grid=(_B,),
        in_specs=[
            pl.BlockSpec((1, _N, _N), lambda b: (b, 0, 0)),
            pl.BlockSpec((1, _N, _N), lambda b: (b, 0, 0)),
            pl.BlockSpec((_N, _NCOL), lambda b: (0, 0)),
            pl.BlockSpec((_N, _NCOL), lambda b: (0, 0)),
            pl.BlockSpec((_N, _N), lambda b: (0, 0)),
            pl.BlockSpec((_N, _N), lambda b: (0, 0)),
            pl.BlockSpec((_N, _N), lambda b: (0, 0)),
        ],
        out_specs=pl.BlockSpec((3, 1, _N, _NCOL), lambda b: (0, b, 0, 0)),
        out_shape=jax.ShapeDtypeStruct((3, _B, _N, _NCOL), jnp.bfloat16),
        compiler_params=pltpu.CompilerParams(
            dimension_semantics=("parallel",),
            vmem_limit_bytes=50 * 1024 * 1024,
        ),
        name="frc_dft_products",
    )(x1, x2, chm, shm, cf, sf, cms)

    prods_flat = prods.reshape(3, _B, _N * _NCOL)
    half = _NP // 2
    part = pl.pallas_call(
        _bin_kernel,
        grid=(2, half),
        in_specs=[
            pl.BlockSpec((1, 1, _PCHUNK), lambda h, p: (h * half + p, 0, 0)),
            pl.BlockSpec((1, 1, _PCHUNK), lambda h, p: (h * half + p, 0, 0)),
            pl.BlockSpec((3, _B, _PCHUNK), lambda h, p: (0, 0, h * half + p)),
        ],
        out_specs=pl.BlockSpec((1, 3 * _B, _NB_PAD), lambda h, p: (h, 0, 0)),
        out_shape=jax.ShapeDtypeStruct((2, 3 * _B, _NB_PAD), jnp.float32),
        compiler_params=pltpu.CompilerParams(
            dimension_semantics=("parallel", "arbitrary"),
            vmem_limit_bytes=50 * 1024 * 1024,
        ),
        name="frc_radial_bins",
    )(idx, w, prods_flat)

    loss = pl.pallas_call(
        _loss_kernel,
        out_shape=jax.ShapeDtypeStruct((1, 1), jnp.float32),
        name="frc_loss_epilogue",
    )(part)
    return loss[0, 0]
